# baseline (speedup 1.0000x reference)
import functools

import jax
import jax.numpy as jnp
from jax import lax
from jax.experimental import pallas as pl
from jax.experimental.pallas import tpu as pltpu
from jax.experimental.pallas import tpu_sc as plsc

TOP_K = 8
EPS = 1e-12


def _mean_norm_body(x_ref, xn_ref, *, ntok):
    mean = jnp.sum(x_ref[...], axis=1, keepdims=True) * (1.0 / ntok)
    sq = jnp.sum(mean * mean, axis=2, keepdims=True)
    inv = lax.rsqrt(jnp.maximum(sq, EPS))
    xn_ref[...] = mean * inv


def _mean_norm(x):
    b, t, d = x.shape
    out = pl.pallas_call(
        functools.partial(_mean_norm_body, ntok=t),
        grid=(b,),
        in_specs=[pl.BlockSpec((1, t, d), lambda i: (i, 0, 0))],
        out_specs=pl.BlockSpec((1, 1, d), lambda i: (i, 0, 0)),
        out_shape=jax.ShapeDtypeStruct((b, 1, d), jnp.float32),
        compiler_params=pltpu.CompilerParams(
            dimension_semantics=("arbitrary",)
        ),
    )(x)
    return out.reshape(b, d)


def _topk_body(xn_ref, p_ref, idx_ref, s_scr, i_scr, *, blk, nblk):
    step = pl.program_id(0)
    block = p_ref[...]
    b = xn_ref.shape[0]

    sqsum = jnp.sum(block * block, axis=1, keepdims=True)
    inv = lax.rsqrt(jnp.maximum(sqsum, EPS))
    bn = (block * inv).astype(jnp.bfloat16)
    xb = xn_ref[...].astype(jnp.bfloat16)
    scores = lax.dot_general(
        xb, bn, (((1,), (1,)), ((), ())),
        preferred_element_type=jnp.float32,
    )

    @pl.when(step == 0)
    def _():
        s_scr[...] = jnp.full_like(s_scr, -jnp.inf)
        i_scr[...] = jnp.zeros_like(i_scr)

    gidx = lax.broadcasted_iota(jnp.int32, (b, blk), 1) + step * blk
    pos = lax.broadcasted_iota(jnp.int32, (b, blk), 1)

    blk_s = []
    blk_i = []
    for _ in range(TOP_K):
        m = jnp.max(scores, axis=1, keepdims=True)
        pk = jnp.min(jnp.where(scores == m, pos, blk), axis=1,
                     keepdims=True)
        sel = pos == pk
        ik = jnp.max(jnp.where(sel, gidx, -1), axis=1, keepdims=True)
        blk_s.append(m)
        blk_i.append(ik)
        scores = jnp.where(sel, -jnp.inf, scores)

    cand_s = jnp.concatenate([s_scr[...]] + blk_s, axis=1)
    cand_i = jnp.concatenate([i_scr[...]] + blk_i, axis=1)
    cpos = lax.broadcasted_iota(jnp.int32, (b, 2 * TOP_K), 1)
    for k in range(TOP_K):
        m = jnp.max(cand_s, axis=1, keepdims=True)
        pk = jnp.min(jnp.where(cand_s == m, cpos, 2 * TOP_K), axis=1,
                     keepdims=True)
        sel = cpos == pk
        ik = jnp.max(jnp.where(sel, cand_i, -1), axis=1)
        s_scr[:, pl.ds(k, 1)] = m
        i_scr[:, pl.ds(k, 1)] = ik[:, None]
        cand_s = jnp.where(sel, -jnp.inf, cand_s)

    @pl.when(step == nblk - 1)
    def _():
        idx_ref[...] = i_scr[...]


def _topk_indices(x_norm, prompt, blk):
    b, d = x_norm.shape
    n = prompt.shape[0]
    nblk = n // blk
    assert nblk * blk == n
    return pl.pallas_call(
        functools.partial(_topk_body, blk=blk, nblk=nblk),
        grid=(nblk,),
        in_specs=[
            pl.BlockSpec((b, d), lambda i: (0, 0)),
            pl.BlockSpec((blk, d), lambda i: (i, 0)),
        ],
        out_specs=pl.BlockSpec((b, TOP_K), lambda i: (0, 0)),
        out_shape=jax.ShapeDtypeStruct((b, TOP_K), jnp.int32),
        scratch_shapes=[
            pltpu.VMEM((b, TOP_K), jnp.float32),
            pltpu.VMEM((b, TOP_K), jnp.int32),
        ],
        compiler_params=pltpu.CompilerParams(
            dimension_semantics=("arbitrary",)
        ),
    )(x_norm, prompt)


def _gather_rows(prompt, idx_flat):
    nrows = idx_flat.shape[0]
    d = prompt.shape[1]
    mesh = plsc.VectorSubcoreMesh(core_axis_name="c", subcore_axis_name="s")

    @functools.partial(
        pl.kernel,
        mesh=mesh,
        out_type=jax.ShapeDtypeStruct((nrows, d), jnp.float32),
        scratch_types=[
            pltpu.VMEM((nrows,), jnp.int32),
            pltpu.VMEM((nrows, d), jnp.float32),
            pltpu.SemaphoreType.DMA,
        ],
    )
    def gather(prompt_hbm, idx_hbm, out_hbm, idx_v, rows_v, sem):
        wid = lax.axis_index("s") * 2 + lax.axis_index("c")

        @pl.when(wid == 0)
        def _():
            pltpu.sync_copy(idx_hbm, idx_v)
            pltpu.async_copy(prompt_hbm.at[idx_v], rows_v, sem).wait()
            pltpu.sync_copy(rows_v, out_hbm)

    return gather(prompt, idx_flat)


def kernel(x, prompt):
    b = x.shape[0]
    x_norm = _mean_norm(x)
    index = _topk_indices(x_norm, prompt, blk=2000)
    p = _gather_rows(prompt, index.reshape(b * TOP_K))
    prompted_embedding = jnp.concatenate(
        [p.reshape(b, TOP_K, prompt.shape[1]), x], axis=1
    )
    return index, prompted_embedding
